# baseline (speedup 1.0000x reference)
import jax
import jax.numpy as jnp
from jax.experimental import pallas as pl
from jax.experimental.pallas import tpu as pltpu

_N = 10000
_IN_CH = 128
_OUT_CH = 64
_CAT = _IN_CH + _OUT_CH
_TILE = 2000


def _fused_tile(x_ref, w0_ref, w1_ref, b_ref, fc1_ref, fc1b_ref, out_ref):
    i = pl.program_id(0)
    X = x_ref[...]
    nrm = jnp.sqrt(jnp.sum(X * X, axis=1, keepdims=True))
    Xn = X / jnp.maximum(nrm, 1e-12)
    P = jnp.dot(Xn, w0_ref[...]) + jnp.dot(Xn, w1_ref[...]) + b_ref[...]
    Z = jax.nn.sigmoid(P[:, :_OUT_CH])
    T = jnp.tanh(P[:, _OUT_CH:])
    H = jnp.maximum((1.0 - Z) * T, 0.0)
    partial = jnp.sum(fc1_ref[...] * H).reshape(1, 1)

    @pl.when(i == 0)
    def _init():
        out_ref[...] = fc1b_ref[...]

    out_ref[...] += partial


def kernel(x, edge_index, edge_weight, Wz, bz, Wr, br, Wh, bh, fc1_w, fc1_b):
    del edge_index, edge_weight, Wr, br
    w0 = jnp.concatenate([Wz[0, 0, :_IN_CH, :], Wh[0, 0, :_IN_CH, :]], axis=1)
    w1 = jnp.concatenate([Wz[1, 0, :_IN_CH, :], Wh[1, 0, :_IN_CH, :]], axis=1)
    b = jnp.concatenate([bz, bh]).reshape(1, 2 * _OUT_CH)
    fc1_m = fc1_w.reshape(_N, _OUT_CH)
    grid = _N // _TILE
    out = pl.pallas_call(
        _fused_tile,
        grid=(grid,),
        in_specs=[
            pl.BlockSpec((_TILE, _IN_CH), lambda i: (i, 0)),
            pl.BlockSpec((_IN_CH, 2 * _OUT_CH), lambda i: (0, 0)),
            pl.BlockSpec((_IN_CH, 2 * _OUT_CH), lambda i: (0, 0)),
            pl.BlockSpec((1, 2 * _OUT_CH), lambda i: (0, 0)),
            pl.BlockSpec((_TILE, _OUT_CH), lambda i: (i, 0)),
            pl.BlockSpec((1, 1), lambda i: (0, 0)),
        ],
        out_specs=pl.BlockSpec((1, 1), lambda i: (0, 0)),
        out_shape=jax.ShapeDtypeStruct((1, 1), jnp.float32),
        compiler_params=pltpu.CompilerParams(
            dimension_semantics=("arbitrary",)),
    )(x, w0, w1, b, fc1_m, fc1_b.reshape(1, 1))
    return out.reshape(1)


# speedup vs baseline: 2.6400x; 2.6400x over previous
import jax
import jax.numpy as jnp
from jax.experimental import pallas as pl
from jax.experimental.pallas import tpu as pltpu

_N = 10000
_IN_CH = 128
_OUT_CH = 64
_CAT = _IN_CH + _OUT_CH
_TILE = 2000


def _fused_tile(x_ref, w0_ref, w1_ref, b_ref, fc1_ref, fc1b_ref, out_ref):
    i = pl.program_id(0)
    X = x_ref[...]
    nrm = jnp.sqrt(jnp.sum(X * X, axis=1, keepdims=True))
    Xn = X / jnp.maximum(nrm, 1e-12)
    P = jnp.dot(Xn, w0_ref[...]) + jnp.dot(Xn, w1_ref[...]) + b_ref[...]
    Z = jax.nn.sigmoid(P[:, :_OUT_CH])
    T = jnp.tanh(P[:, _OUT_CH:])
    H = jnp.maximum((1.0 - Z) * T, 0.0)
    H3 = H.reshape(_TILE // 2, 2, _OUT_CH)
    H2 = jnp.concatenate([H3[:, 0, :], H3[:, 1, :]], axis=-1)
    partial = jnp.sum(fc1_ref[...] * H2).reshape(1, 1)

    @pl.when(i == 0)
    def _init():
        out_ref[...] = fc1b_ref[...]

    out_ref[...] += partial


def kernel(x, edge_index, edge_weight, Wz, bz, Wr, br, Wh, bh, fc1_w, fc1_b):
    del edge_index, edge_weight, Wr, br
    w0 = jnp.concatenate([Wz[0, 0, :_IN_CH, :], Wh[0, 0, :_IN_CH, :]], axis=1)
    w1 = jnp.concatenate([Wz[1, 0, :_IN_CH, :], Wh[1, 0, :_IN_CH, :]], axis=1)
    b = jnp.concatenate([bz, bh]).reshape(1, 2 * _OUT_CH)
    fc1_p = fc1_w.reshape(_N // 2, 2 * _OUT_CH)
    grid = _N // _TILE
    out = pl.pallas_call(
        _fused_tile,
        grid=(grid,),
        in_specs=[
            pl.BlockSpec((_TILE, _IN_CH), lambda i: (i, 0)),
            pl.BlockSpec((_IN_CH, 2 * _OUT_CH), lambda i: (0, 0)),
            pl.BlockSpec((_IN_CH, 2 * _OUT_CH), lambda i: (0, 0)),
            pl.BlockSpec((1, 2 * _OUT_CH), lambda i: (0, 0)),
            pl.BlockSpec((_TILE // 2, 2 * _OUT_CH), lambda i: (i, 0)),
            pl.BlockSpec((1, 1), lambda i: (0, 0)),
        ],
        out_specs=pl.BlockSpec((1, 1), lambda i: (0, 0)),
        out_shape=jax.ShapeDtypeStruct((1, 1), jnp.float32),
        compiler_params=pltpu.CompilerParams(
            dimension_semantics=("arbitrary",)),
    )(x, w0, w1, b, fc1_p, fc1_b.reshape(1, 1))
    return out.reshape(1)
